# baseline (speedup 1.0000x reference)
import jax
import jax.numpy as jnp
from jax.experimental import pallas as pl
from jax.experimental.pallas import tpu as pltpu

_N = 10000
_D = 128
_BM1 = 400
_BM2 = 1000


def _layer1_kernel(a_ref, x_ref, w1_ref, b_ref, w_ref, pn_ref, abf_ref, p1_ref):
    i = pl.program_id(0)

    @pl.when(i == 0)
    def _():
        p1_ref[...] = jnp.dot(
            x_ref[...].astype(jnp.bfloat16),
            w1_ref[...],
            preferred_element_type=jnp.float32,
        ).astype(jnp.bfloat16)

    a16 = a_ref[...].astype(jnp.bfloat16)
    abf_ref[...] = a16
    h = jnp.dot(a16, p1_ref[...], preferred_element_type=jnp.float32) + b_ref[...]
    h = jnp.maximum(h, 0.0).astype(jnp.bfloat16)
    pn_ref[...] = jnp.dot(
        h, w_ref[...], preferred_element_type=jnp.float32
    ).astype(jnp.bfloat16)


def _layer1(adj, x, w1, b, w):
    ni = _N // _BM1
    return pl.pallas_call(
        _layer1_kernel,
        grid=(ni,),
        in_specs=[
            pl.BlockSpec((_BM1, _N), lambda i: (i, 0)),
            pl.BlockSpec((_N, _D), lambda i: (0, 0)),
            pl.BlockSpec((_D, _D), lambda i: (0, 0)),
            pl.BlockSpec((1, _D), lambda i: (0, 0)),
            pl.BlockSpec((_D, _D), lambda i: (0, 0)),
        ],
        out_specs=[
            pl.BlockSpec((_BM1, _D), lambda i: (i, 0)),
            pl.BlockSpec((_BM1, _N), lambda i: (i, 0)),
        ],
        out_shape=[
            jax.ShapeDtypeStruct((_N, _D), jnp.bfloat16),
            jax.ShapeDtypeStruct((_N, _N), jnp.bfloat16),
        ],
        scratch_shapes=[pltpu.VMEM((_N, _D), jnp.bfloat16)],
        compiler_params=pltpu.CompilerParams(
            dimension_semantics=("arbitrary",),
            vmem_limit_bytes=64 * 1024 * 1024,
        ),
    )(adj, x, w1.astype(jnp.bfloat16), b, w.astype(jnp.bfloat16))


def _tail_kernel(a_ref, p2_ref, b2_ref, w3_ref, b3_ref, o_ref, p3_ref):
    l = pl.program_id(0)
    i = pl.program_id(1)

    @pl.when(l == 0)
    def _():
        h = jnp.dot(a_ref[...], p2_ref[...], preferred_element_type=jnp.float32)
        h = jnp.maximum(h + b2_ref[...], 0.0).astype(jnp.bfloat16)
        p3_ref[pl.ds(i * _BM2, _BM2), :] = jnp.dot(
            h, w3_ref[...], preferred_element_type=jnp.float32
        ).astype(jnp.bfloat16)

    @pl.when(l == 1)
    def _():
        o_ref[...] = (
            jnp.dot(a_ref[...], p3_ref[...], preferred_element_type=jnp.float32)
            + b3_ref[...]
        )


def _tail(abf, p2, b2, w3, b3):
    ni = _N // _BM2
    return pl.pallas_call(
        _tail_kernel,
        grid=(2, ni),
        in_specs=[
            pl.BlockSpec((_BM2, _N), lambda l, i: (i, 0)),
            pl.BlockSpec((_N, _D), lambda l, i: (0, 0)),
            pl.BlockSpec((1, _D), lambda l, i: (0, 0)),
            pl.BlockSpec((_D, _D), lambda l, i: (0, 0)),
            pl.BlockSpec((1, _D), lambda l, i: (0, 0)),
        ],
        out_specs=pl.BlockSpec(
            (_BM2, _D), lambda l, i: (i, 0),
            pipeline_mode=pl.Buffered(buffer_count=1, revisit=pl.RevisitMode.ANY),
        ),
        out_shape=jax.ShapeDtypeStruct((_N, _D), jnp.float32),
        scratch_shapes=[pltpu.VMEM((_N, _D), jnp.bfloat16)],
        compiler_params=pltpu.CompilerParams(
            dimension_semantics=("arbitrary", "arbitrary"),
        ),
    )(abf, p2, b2, w3.astype(jnp.bfloat16), b3)


def kernel(x, adj_matrix, W1, b1, W2, b2, W3, b3):
    b1 = b1.reshape(1, _D)
    b2 = b2.reshape(1, _D)
    b3 = b3.reshape(1, _D)
    p2, abf = _layer1(adj_matrix, x, W1, b1, W2)
    return _tail(abf, p2, b2, W3, b3)


# speedup vs baseline: 1.0014x; 1.0014x over previous
import jax
import jax.numpy as jnp
from jax.experimental import pallas as pl
from jax.experimental.pallas import tpu as pltpu

_N = 10000
_D = 128
_BM1 = 400
_BM2 = 1000


def _layer1_kernel(a_ref, x_ref, w1_ref, b_ref, w_ref, pn_ref, abf_ref, p1_ref):
    i = pl.program_id(0)

    @pl.when(i == 0)
    def _():
        p1_ref[...] = jnp.dot(
            x_ref[...].astype(jnp.bfloat16),
            w1_ref[...],
            preferred_element_type=jnp.float32,
        ).astype(jnp.bfloat16)

    a16 = a_ref[...].astype(jnp.bfloat16)
    abf_ref[...] = a16
    h = jnp.dot(a16, p1_ref[...], preferred_element_type=jnp.float32) + b_ref[...]
    h = jnp.maximum(h, 0.0).astype(jnp.bfloat16)
    pn_ref[...] = jnp.dot(
        h, w_ref[...], preferred_element_type=jnp.float32
    ).astype(jnp.bfloat16)


def _layer1(adj, x, w1, b, w):
    ni = _N // _BM1
    return pl.pallas_call(
        _layer1_kernel,
        grid=(ni,),
        in_specs=[
            pl.BlockSpec((_BM1, _N), lambda i: (i, 0)),
            pl.BlockSpec((_N, _D), lambda i: (0, 0)),
            pl.BlockSpec((_D, _D), lambda i: (0, 0)),
            pl.BlockSpec((1, _D), lambda i: (0, 0)),
            pl.BlockSpec((_D, _D), lambda i: (0, 0)),
        ],
        out_specs=[
            pl.BlockSpec((_BM1, _D), lambda i: (i, 0)),
            pl.BlockSpec((_BM1, _N), lambda i: (i, 0)),
        ],
        out_shape=[
            jax.ShapeDtypeStruct((_N, _D), jnp.bfloat16),
            jax.ShapeDtypeStruct((_N, _N), jnp.bfloat16),
        ],
        scratch_shapes=[pltpu.VMEM((_N, _D), jnp.bfloat16)],
        compiler_params=pltpu.CompilerParams(
            dimension_semantics=("arbitrary",),
            vmem_limit_bytes=64 * 1024 * 1024,
        ),
    )(adj, x, w1.astype(jnp.bfloat16), b, w.astype(jnp.bfloat16))


_BMT = 400
_NSLOT = 3


def _tail_kernel(abf_hbm, p2_ref, b2_ref, w3_ref, b3_ref, o_ref, p3_ref,
                 abuf, sem):
    ni = _N // _BMT
    l = pl.program_id(0)
    i = pl.program_id(1)
    s = l * ni + i

    def _copy(t, slot):
        return pltpu.make_async_copy(
            abf_hbm.at[pl.ds((t % ni) * _BMT, _BMT), :],
            abuf.at[slot],
            sem.at[slot],
        )

    @pl.when(s == 0)
    def _():
        for k in range(_NSLOT):
            _copy(k, k).start()

    @pl.when((s >= 1) & (s + _NSLOT - 1 < 2 * ni))
    def _():
        for k in range(_NSLOT):
            @pl.when((s + _NSLOT - 1) % _NSLOT == k)
            def _(k=k):
                _copy(s + _NSLOT - 1, k).start()

    for k in range(_NSLOT):
        @pl.when(s % _NSLOT == k)
        def _(k=k):
            _copy(s, k).wait()
            a = abuf[k]

            @pl.when(l == 0)
            def _():
                h = jnp.dot(a, p2_ref[...], preferred_element_type=jnp.float32)
                h = jnp.maximum(h + b2_ref[...], 0.0).astype(jnp.bfloat16)
                p3_ref[pl.ds(i * _BMT, _BMT), :] = jnp.dot(
                    h, w3_ref[...], preferred_element_type=jnp.float32
                ).astype(jnp.bfloat16)

            @pl.when(l == 1)
            def _():
                o_ref[...] = (
                    jnp.dot(a, p3_ref[...], preferred_element_type=jnp.float32)
                    + b3_ref[...]
                )


def _tail(abf, p2, b2, w3, b3):
    ni = _N // _BMT
    return pl.pallas_call(
        _tail_kernel,
        grid=(2, ni),
        in_specs=[
            pl.BlockSpec(memory_space=pl.ANY),
            pl.BlockSpec((_N, _D), lambda l, i: (0, 0)),
            pl.BlockSpec((1, _D), lambda l, i: (0, 0)),
            pl.BlockSpec((_D, _D), lambda l, i: (0, 0)),
            pl.BlockSpec((1, _D), lambda l, i: (0, 0)),
        ],
        out_specs=pl.BlockSpec(
            (_BMT, _D),
            lambda l, i: (jnp.where(l == 0, ni, i), 0),
        ),
        out_shape=jax.ShapeDtypeStruct((_N + _BMT, _D), jnp.float32),
        scratch_shapes=[
            pltpu.VMEM((_N, _D), jnp.bfloat16),
            pltpu.VMEM((_NSLOT, _BMT, _N), jnp.bfloat16),
            pltpu.SemaphoreType.DMA((_NSLOT,)),
        ],
        compiler_params=pltpu.CompilerParams(
            dimension_semantics=("arbitrary", "arbitrary"),
        ),
    )(abf, p2, b2, w3.astype(jnp.bfloat16), b3)


def kernel(x, adj_matrix, W1, b1, W2, b2, W3, b3):
    b1 = b1.reshape(1, _D)
    b2 = b2.reshape(1, _D)
    b3 = b3.reshape(1, _D)
    p2, abf = _layer1(adj_matrix, x, W1, b1, W2)
    return _tail(abf, p2, b2, W3, b3)[:_N]


# speedup vs baseline: 1.0856x; 1.0841x over previous
import jax
import jax.numpy as jnp
from jax.experimental import pallas as pl
from jax.experimental.pallas import tpu as pltpu

_N = 10000
_D = 128
_BM1 = 400
_BM2 = 1000


def _layer1_kernel(a_ref, x_ref, w1_ref, b_ref, w_ref, pn_ref, abf_ref, p1_ref):
    i = pl.program_id(0)

    @pl.when(i == 0)
    def _():
        p1_ref[...] = jnp.dot(
            x_ref[...].astype(jnp.bfloat16),
            w1_ref[...],
            preferred_element_type=jnp.float32,
        ).astype(jnp.bfloat16)

    a16 = a_ref[...].astype(jnp.bfloat16)
    abf_ref[...] = a16
    h = jnp.dot(a16, p1_ref[...], preferred_element_type=jnp.float32) + b_ref[...]
    h = jnp.maximum(h, 0.0).astype(jnp.bfloat16)
    pn_ref[...] = jnp.dot(
        h, w_ref[...], preferred_element_type=jnp.float32
    ).astype(jnp.bfloat16)


def _layer1(adj, x, w1, b, w):
    ni = _N // _BM1
    return pl.pallas_call(
        _layer1_kernel,
        grid=(ni,),
        in_specs=[
            pl.BlockSpec((_BM1, _N), lambda i: (i, 0)),
            pl.BlockSpec((_N, _D), lambda i: (0, 0)),
            pl.BlockSpec((_D, _D), lambda i: (0, 0)),
            pl.BlockSpec((1, _D), lambda i: (0, 0)),
            pl.BlockSpec((_D, _D), lambda i: (0, 0)),
        ],
        out_specs=[
            pl.BlockSpec((_BM1, _D), lambda i: (i, 0)),
            pl.BlockSpec((_BM1, _N), lambda i: (i, 0)),
        ],
        out_shape=[
            jax.ShapeDtypeStruct((_N, _D), jnp.bfloat16),
            jax.ShapeDtypeStruct((_N, _N), jnp.bfloat16),
        ],
        scratch_shapes=[pltpu.VMEM((_N, _D), jnp.bfloat16)],
        compiler_params=pltpu.CompilerParams(
            dimension_semantics=("arbitrary",),
            vmem_limit_bytes=64 * 1024 * 1024,
        ),
    )(adj, x, w1.astype(jnp.bfloat16), b, w.astype(jnp.bfloat16))


def _tail_kernel(a_ref, p2_ref, b2_ref, w3_ref, b3_ref, o_ref, p3_ref):
    l = pl.program_id(0)
    i = pl.program_id(1)

    @pl.when(l == 0)
    def _():
        h = jnp.dot(a_ref[...], p2_ref[...], preferred_element_type=jnp.float32)
        h = jnp.maximum(h + b2_ref[...], 0.0).astype(jnp.bfloat16)
        p3_ref[pl.ds(i * _BM2, _BM2), :] = jnp.dot(
            h, w3_ref[...], preferred_element_type=jnp.float32
        ).astype(jnp.bfloat16)

    @pl.when(l == 1)
    def _():
        o_ref[...] = (
            jnp.dot(a_ref[...], p3_ref[...], preferred_element_type=jnp.float32)
            + b3_ref[...]
        )


def _tail(abf, p2, b2, w3, b3):
    ni = _N // _BM2
    return pl.pallas_call(
        _tail_kernel,
        grid=(2, ni),
        in_specs=[
            pl.BlockSpec((_BM2, _N), lambda l, i: (i, 0)),
            pl.BlockSpec((_N, _D), lambda l, i: (0, 0)),
            pl.BlockSpec((1, _D), lambda l, i: (0, 0)),
            pl.BlockSpec((_D, _D), lambda l, i: (0, 0)),
            pl.BlockSpec((1, _D), lambda l, i: (0, 0)),
        ],
        out_specs=pl.BlockSpec(
            (_BM2, _D),
            lambda l, i: (jnp.where(l == 0, ni, i), 0),
        ),
        out_shape=jax.ShapeDtypeStruct((_N + _BM2, _D), jnp.float32),
        scratch_shapes=[pltpu.VMEM((_N, _D), jnp.bfloat16)],
        compiler_params=pltpu.CompilerParams(
            dimension_semantics=("arbitrary", "arbitrary"),
        ),
    )(abf, p2, b2, w3.astype(jnp.bfloat16), b3)


def kernel(x, adj_matrix, W1, b1, W2, b2, W3, b3):
    b1 = b1.reshape(1, _D)
    b2 = b2.reshape(1, _D)
    b3 = b3.reshape(1, _D)
    p2, abf = _layer1(adj_matrix, x, W1, b1, W2)
    return _tail(abf, p2, b2, W3, b3)[:_N]
